# baseline (device time: 32706 ns/iter reference)
import jax
import jax.numpy as jnp
from jax import lax
from jax.experimental import pallas as pl
from jax.experimental.pallas import tpu as pltpu


def kernel(partial, resid, gamma):
    m, d = resid.shape
    p = partial.reshape(m, d).astype(jnp.bfloat16)
    g = gamma.reshape(1, d)

    def body(p_ref, resid_ref, g_ref, out_ref, recv_buf, send_sem, recv_sem):
        my_x = lax.axis_index("x")
        my_y = lax.axis_index("y")
        my_z = lax.axis_index("z")
        nbr = (1 - my_x, my_y, my_z)

        barrier_sem = pltpu.get_barrier_semaphore()
        pl.semaphore_signal(
            barrier_sem, inc=1, device_id=nbr,
            device_id_type=pl.DeviceIdType.MESH,
        )
        pl.semaphore_wait(barrier_sem, 1)

        rdma = pltpu.make_async_remote_copy(
            src_ref=p_ref,
            dst_ref=recv_buf,
            send_sem=send_sem,
            recv_sem=recv_sem,
            device_id=nbr,
            device_id_type=pl.DeviceIdType.MESH,
        )
        rdma.start()
        rdma.wait()

        y = (
            p_ref[...].astype(jnp.float32)
            + recv_buf[...].astype(jnp.float32)
            + resid_ref[...]
        )
        rms = jnp.sqrt(jnp.mean(y * y, axis=-1, keepdims=True) + 1e-6)
        out_ref[...] = y / rms * g_ref[...]

    return pl.pallas_call(
        body,
        out_shape=jax.ShapeDtypeStruct((m, d), jnp.float32),
        in_specs=[
            pl.BlockSpec(memory_space=pltpu.VMEM),
            pl.BlockSpec(memory_space=pltpu.VMEM),
            pl.BlockSpec(memory_space=pltpu.VMEM),
        ],
        out_specs=pl.BlockSpec(memory_space=pltpu.VMEM),
        scratch_shapes=[
            pltpu.VMEM((m, d), jnp.bfloat16),
            pltpu.SemaphoreType.DMA,
            pltpu.SemaphoreType.DMA,
        ],
        compiler_params=pltpu.CompilerParams(collective_id=0),
    )(p, resid, g)


# device time: 24221 ns/iter; 1.3503x vs baseline; 1.3503x over previous
import jax
import jax.numpy as jnp
from jax import lax
from jax.experimental import pallas as pl
from jax.experimental.pallas import tpu as pltpu

NC = 8


def kernel(partial, resid, gamma):
    m, d = resid.shape
    half = m // 2
    rows = half // NC
    p = partial.reshape(m, d).astype(jnp.bfloat16)
    g = gamma.reshape(1, d)

    def body(
        p_ref, resid_ref, g_ref, out_ref,
        prbuf, ybuf, orbuf,
        x_send_sems, x_recv_sems, y_send_sems, y_recv_sems,
    ):
        my_x = lax.axis_index("x")
        my_y = lax.axis_index("y")
        my_z = lax.axis_index("z")
        xn = (1 - my_x, my_y, my_z)
        yn = (my_x, 1 - my_y, my_z)
        h = my_x ^ my_y
        my0 = h * half
        ot0 = (1 - h) * half

        barrier_sem = pltpu.get_barrier_semaphore()
        for nbr in (xn, yn):
            pl.semaphore_signal(
                barrier_sem, inc=1, device_id=nbr,
                device_id_type=pl.DeviceIdType.MESH,
            )
        pl.semaphore_wait(barrier_sem, 2)

        x_rdmas = []
        for c in range(NC):
            rd = pltpu.make_async_remote_copy(
                src_ref=p_ref.at[pl.ds(ot0 + c * rows, rows), :],
                dst_ref=prbuf.at[c],
                send_sem=x_send_sems.at[c],
                recv_sem=x_recv_sems.at[c],
                device_id=xn,
                device_id_type=pl.DeviceIdType.MESH,
            )
            rd.start()
            x_rdmas.append(rd)

        y_rdmas = []
        for c in range(NC):
            x_rdmas[c].wait_recv()
            r0 = my0 + c * rows
            yv = (
                p_ref[pl.ds(r0, rows), :].astype(jnp.float32)
                + prbuf[c].astype(jnp.float32)
                + resid_ref[pl.ds(r0, rows), :]
            )
            rms = jnp.sqrt(jnp.mean(yv * yv, axis=-1, keepdims=True) + 1e-6)
            o = yv / rms * g_ref[...]
            out_ref[pl.ds(r0, rows), :] = o
            ybuf[c] = o.astype(jnp.bfloat16)
            rd = pltpu.make_async_remote_copy(
                src_ref=ybuf.at[c],
                dst_ref=orbuf.at[c],
                send_sem=y_send_sems.at[c],
                recv_sem=y_recv_sems.at[c],
                device_id=yn,
                device_id_type=pl.DeviceIdType.MESH,
            )
            rd.start()
            y_rdmas.append(rd)

        for c in range(NC):
            y_rdmas[c].wait_recv()
            out_ref[pl.ds(ot0 + c * rows, rows), :] = orbuf[c].astype(
                jnp.float32
            )

        for c in range(NC):
            x_rdmas[c].wait_send()
            y_rdmas[c].wait_send()

    return pl.pallas_call(
        body,
        out_shape=jax.ShapeDtypeStruct((m, d), jnp.float32),
        in_specs=[
            pl.BlockSpec(memory_space=pltpu.VMEM),
            pl.BlockSpec(memory_space=pltpu.VMEM),
            pl.BlockSpec(memory_space=pltpu.VMEM),
        ],
        out_specs=pl.BlockSpec(memory_space=pltpu.VMEM),
        scratch_shapes=[
            pltpu.VMEM((NC, rows, d), jnp.bfloat16),
            pltpu.VMEM((NC, rows, d), jnp.bfloat16),
            pltpu.VMEM((NC, rows, d), jnp.bfloat16),
            pltpu.SemaphoreType.DMA((NC,)),
            pltpu.SemaphoreType.DMA((NC,)),
            pltpu.SemaphoreType.DMA((NC,)),
            pltpu.SemaphoreType.DMA((NC,)),
        ],
        compiler_params=pltpu.CompilerParams(collective_id=0),
    )(p, resid, g)
